# Initial kernel scaffold; baseline (speedup 1.0000x reference)
import jax
import jax.numpy as jnp
from jax.experimental import pallas as pl


def kernel(x, embed_weight):
    raise NotImplementedError("write your pallas kernel here")


# speedup vs baseline: 1.3592x; 1.3592x over previous
import functools

import jax
import jax.numpy as jnp
from jax import lax
from jax.experimental import pallas as pl
from jax.experimental.pallas import tpu as pltpu
from jax.experimental.pallas import tpu_sc as plsc

_NC = 2
_NS = 16
_NW = _NC * _NS


def _broadcast_table(w, b):
    t, d = w.shape
    rows = t // _NW

    mesh = plsc.VectorSubcoreMesh(core_axis_name="c", subcore_axis_name="s")

    @functools.partial(
        pl.kernel,
        mesh=mesh,
        out_type=jax.ShapeDtypeStruct((b, t, d), jnp.float32),
        scratch_types=[
            pltpu.VMEM((rows, d), jnp.float32),
        ] + [pltpu.SemaphoreType.DMA] * b,
    )
    def k(w_hbm, out_hbm, buf, *sems):
        wid = lax.axis_index("s") * _NC + lax.axis_index("c")
        base = wid * rows
        pltpu.sync_copy(w_hbm.at[pl.ds(base, rows)], buf)
        copies = [
            pltpu.async_copy(buf, out_hbm.at[i, pl.ds(base, rows)], sems[i])
            for i in range(b)
        ]
        for c in copies:
            c.wait()

    return k(w)


def kernel(x, embed_weight):
    b = x.shape[0]
    return _broadcast_table(embed_weight, b)
